# baseline (speedup 1.0000x reference)
import functools

import jax
import jax.numpy as jnp
from jax import lax
from jax.experimental import pallas as pl
from jax.experimental.pallas import tpu as pltpu
from jax.experimental.pallas import tpu_sc as plsc

N, D, B, R, REL_D, NE = 10000, 128, 4096, 200, 64, 10000

NEG = -1e30

_NC, _NS = 2, 16
_NW = _NC * _NS
_BPW = B // _NW


def _sc_body(static_hbm, dynamic_hbm, relcat_hbm,
             eh_hbm, er_hbm,
             hs_out, hd_out, rr_out,
             idxh_v, idxr_v, rows_s, rows_d, rows_r,
             sem1, sem2, sem3):
    wid = lax.axis_index("s") * _NC + lax.axis_index("c")
    base = wid * _BPW
    pltpu.sync_copy(eh_hbm.at[pl.ds(base, _BPW)], idxh_v)
    pltpu.sync_copy(er_hbm.at[pl.ds(base, _BPW)], idxr_v)
    c1 = pltpu.async_copy(static_hbm.at[idxh_v], rows_s, sem1)
    c2 = pltpu.async_copy(dynamic_hbm.at[idxh_v], rows_d, sem2)
    c3 = pltpu.async_copy(relcat_hbm.at[idxr_v], rows_r, sem3)
    c1.wait()
    c2.wait()
    c3.wait()
    pltpu.sync_copy(rows_s, hs_out.at[pl.ds(base, _BPW)])
    pltpu.sync_copy(rows_d, hd_out.at[pl.ds(base, _BPW)])
    pltpu.sync_copy(rows_r, rr_out.at[pl.ds(base, _BPW)])


def _sc_gather(static_emb, dynamic_emb, relcat, edge_head, edge_rels):
    mesh = plsc.VectorSubcoreMesh(core_axis_name="c", subcore_axis_name="s")
    f = functools.partial(
        pl.kernel,
        mesh=mesh,
        out_type=[
            jax.ShapeDtypeStruct((B, D), jnp.float32),
            jax.ShapeDtypeStruct((B, D), jnp.float32),
            jax.ShapeDtypeStruct((B, 2 * REL_D), jnp.float32),
        ],
        scratch_types=[
            pltpu.VMEM((_BPW,), jnp.int32),
            pltpu.VMEM((_BPW,), jnp.int32),
            pltpu.VMEM((_BPW, D), jnp.float32),
            pltpu.VMEM((_BPW, D), jnp.float32),
            pltpu.VMEM((_BPW, 2 * REL_D), jnp.float32),
            pltpu.SemaphoreType.DMA,
            pltpu.SemaphoreType.DMA,
            pltpu.SemaphoreType.DMA,
        ],
    )(_sc_body)
    return f(static_emb, dynamic_emb, relcat, edge_head, edge_rels)


CHUNK = 1024
T = (NE + CHUNK - 1) // CHUNK
NE_PAD = T * CHUNK
SUB = 512
NSUB = B // SUB


def _main_body(static_ref, hs_ref, hd_ref, rr_ref,
               wh1_ref, bh1_ref, wh2_ref, bh2_ref,
               wr1_ref, br1_ref, wr2_ref, br2_ref,
               wt1_ref, bt1_ref, wt2_ref, bt2_ref,
               er_ref, et_ref, eh_ref,
               out_ref,
               hidden_scr, h1_scr, s_scr, tgt_scr,
               rel_scr, hsum_scr, shead_scr):
    j = pl.program_id(0)

    @pl.when(j == 0)
    def _prologue():
        gemb = jnp.max(static_ref[...], axis=0, keepdims=True)
        h1_scr[...] = jnp.tanh(
            jnp.dot(gemb, wh1_ref[...], preferred_element_type=jnp.float32)
            + bh1_ref[...])
        br1_eff = br1_ref[...] + jnp.dot(
            gemb, wr1_ref[pl.ds(2 * D, D), :],
            preferred_element_type=jnp.float32)
        bt1_eff = bt1_ref[...] + jnp.dot(
            gemb, wt1_ref[pl.ds(2 * D, D), :],
            preferred_element_type=jnp.float32)
        wr1s = wr1_ref[pl.ds(0, D), :].astype(jnp.bfloat16)
        wr1d = wr1_ref[pl.ds(D, D), :].astype(jnp.bfloat16)
        wr2 = wr2_ref[...].astype(jnp.bfloat16)
        wt1s = wt1_ref[pl.ds(0, D), :].astype(jnp.bfloat16)
        wt1d = wt1_ref[pl.ds(D, D), :].astype(jnp.bfloat16)
        wt1r = wt1_ref[pl.ds(3 * D, 2 * REL_D), :].astype(jnp.bfloat16)
        rel_sum = jnp.float32(0.0)
        for bi in range(NSUB):
            sl = pl.ds(bi * SUB, SUB)
            hs = hs_ref[sl, :].astype(jnp.bfloat16)
            hd = hd_ref[sl, :].astype(jnp.bfloat16)
            relh = jnp.tanh(
                jnp.dot(hs, wr1s, preferred_element_type=jnp.float32)
                + jnp.dot(hd, wr1d, preferred_element_type=jnp.float32)
                + br1_eff)
            rl = (jnp.dot(relh.astype(jnp.bfloat16), wr2,
                          preferred_element_type=jnp.float32) + br2_ref[...])
            lser = jnp.log(jnp.sum(jnp.exp(rl), axis=1, keepdims=True))
            colr = lax.broadcasted_iota(jnp.int32, rl.shape, 1)
            tgtr = jnp.sum(jnp.where(colr == er_ref[sl, :], rl, 0.0),
                           axis=1, keepdims=True)
            rel_sum = rel_sum + jnp.sum(tgtr - lser)
            th = jnp.tanh(
                jnp.dot(hs, wt1s, preferred_element_type=jnp.float32)
                + jnp.dot(hd, wt1d, preferred_element_type=jnp.float32)
                + jnp.dot(rr_ref[sl, :].astype(jnp.bfloat16), wt1r,
                          preferred_element_type=jnp.float32)
                + bt1_eff)
            hidden_scr[sl, :] = th.astype(jnp.bfloat16)
        rel_scr[0, 0] = rel_sum
        hsum_scr[0, 0] = 0.0
        shead_scr[0, 0] = 0.0
        s_scr[...] = jnp.zeros((B, 1), jnp.float32)
        tgt_scr[...] = jnp.zeros((B, 1), jnp.float32)

    hl = (jnp.dot(h1_scr[...].astype(jnp.bfloat16),
                  wh2_ref[...].astype(jnp.bfloat16),
                  preferred_element_type=jnp.float32) + bh2_ref[...])
    colbase = j * CHUNK
    is_last = j == T - 1

    def _mask_tail(x):
        c = colbase + lax.broadcasted_iota(jnp.int32, x.shape, 1)
        return jnp.where(c < NE, x, NEG)

    hl = lax.cond(is_last, _mask_tail, lambda x: x, hl)
    shead_scr[0, 0] = shead_scr[0, 0] + jnp.sum(jnp.exp(hl))

    wt2 = wt2_ref[...].astype(jnp.bfloat16)
    bt2 = bt2_ref[...]
    hsum = jnp.float32(0.0)
    for bi in range(NSUB):
        sl = pl.ds(bi * SUB, SUB)
        h = hidden_scr[sl, :]
        lg = (jnp.dot(h, wt2, preferred_element_type=jnp.float32) + bt2)
        lg = lax.cond(is_last, _mask_tail, lambda x: x, lg)
        col = colbase + lax.broadcasted_iota(jnp.int32, lg.shape, 1)
        s_scr[sl, :] = s_scr[sl, :] + jnp.sum(jnp.exp(lg), axis=1,
                                              keepdims=True)
        tgt_scr[sl, :] = tgt_scr[sl, :] + jnp.sum(
            jnp.where(col == et_ref[sl, :], lg, 0.0), axis=1, keepdims=True)
        hsum = hsum + jnp.sum(jnp.where(col == eh_ref[sl, :], hl, 0.0))
    hsum_scr[0, 0] = hsum_scr[0, 0] + hsum

    @pl.when(j == T - 1)
    def _epilogue():
        lse_t = jnp.log(s_scr[...])
        lp_tail = jnp.sum(tgt_scr[...] - lse_t) / B
        lp_rel = rel_scr[0, 0] / B
        lp_head = hsum_scr[0, 0] / B - jnp.log(shead_scr[0, 0])
        out_ref[0, 0] = lp_head + lp_rel + lp_tail


def _main_branch(static_emb, hs, hd, rr, Wh1, bh1, Wh2, bh2,
                 Wr1, br1, Wr2, br2, Wt1, bt1, Wt2, bt2,
                 edge_rels, edge_tail, edge_head):
    call = pl.pallas_call(
        _main_body,
        grid=(T,),
        in_specs=[
            pl.BlockSpec((N, D), lambda j: (0, 0)),
            pl.BlockSpec((B, D), lambda j: (0, 0)),
            pl.BlockSpec((B, D), lambda j: (0, 0)),
            pl.BlockSpec((B, 2 * REL_D), lambda j: (0, 0)),
            pl.BlockSpec((D, 4 * D), lambda j: (0, 0)),
            pl.BlockSpec((1, 4 * D), lambda j: (0, 0)),
            pl.BlockSpec((4 * D, CHUNK), lambda j: (0, j)),
            pl.BlockSpec((1, CHUNK), lambda j: (0, j)),
            pl.BlockSpec((3 * D, 3 * D), lambda j: (0, 0)),
            pl.BlockSpec((1, 3 * D), lambda j: (0, 0)),
            pl.BlockSpec((3 * D, R), lambda j: (0, 0)),
            pl.BlockSpec((1, R), lambda j: (0, 0)),
            pl.BlockSpec((4 * D, 2 * 4 * D), lambda j: (0, 0)),
            pl.BlockSpec((1, 2 * 4 * D), lambda j: (0, 0)),
            pl.BlockSpec((2 * 4 * D, CHUNK), lambda j: (0, j)),
            pl.BlockSpec((1, CHUNK), lambda j: (0, j)),
            pl.BlockSpec((B, 1), lambda j: (0, 0)),
            pl.BlockSpec((B, 1), lambda j: (0, 0)),
            pl.BlockSpec((B, 1), lambda j: (0, 0)),
        ],
        out_specs=pl.BlockSpec(memory_space=pltpu.SMEM),
        out_shape=jax.ShapeDtypeStruct((1, 1), jnp.float32),
        scratch_shapes=[
            pltpu.VMEM((B, 8 * D), jnp.bfloat16),
            pltpu.VMEM((1, 4 * D), jnp.float32),
            pltpu.VMEM((B, 1), jnp.float32),
            pltpu.VMEM((B, 1), jnp.float32),
            pltpu.SMEM((1, 1), jnp.float32),
            pltpu.SMEM((1, 1), jnp.float32),
            pltpu.SMEM((1, 1), jnp.float32),
        ],
        compiler_params=pltpu.CompilerParams(
            dimension_semantics=("arbitrary",)),
    )
    bh2p = jnp.pad(bh2.reshape(1, -1), ((0, 0), (0, NE_PAD - NE)),
                   constant_values=NEG)
    bt2p = jnp.pad(bt2.reshape(1, -1), ((0, 0), (0, NE_PAD - NE)),
                   constant_values=NEG)
    out = call(static_emb, hs, hd, rr,
               Wh1, bh1.reshape(1, -1), Wh2, bh2p,
               Wr1, br1.reshape(1, -1), Wr2, br2.reshape(1, -1),
               Wt1, bt1.reshape(1, -1), Wt2, bt2p,
               edge_rels.reshape(B, 1), edge_tail.reshape(B, 1),
               edge_head.reshape(B, 1))
    return out


def kernel(static_emb, dynamic_emb, dynamic_relation_emb, rel_embeds,
           Wh1, bh1, Wh2, bh2, Wr1, br1, Wr2, br2, Wt1, bt1, Wt2, bt2,
           node_ids, edge_head, edge_rels, edge_tail):
    relcat = jnp.concatenate([rel_embeds, dynamic_relation_emb], axis=1)
    hs, hd, rr = _sc_gather(static_emb, dynamic_emb, relcat,
                            edge_head, edge_rels)
    out = _main_branch(static_emb, hs, hd, rr, Wh1, bh1, Wh2, bh2,
                       Wr1, br1, Wr2, br2, Wt1, bt1, Wt2, bt2,
                       edge_rels, edge_tail, edge_head)
    return out[0, 0]
